# Initial kernel scaffold; baseline (speedup 1.0000x reference)
import jax
import jax.numpy as jnp
from jax.experimental import pallas as pl


def kernel(userIdx, servIdx, eu_gmf, eu_mlp, ei_gmf, ei_mlp, W1, b1, W2, b2, W3, b3, Wp, bp):
    raise NotImplementedError("write your pallas kernel here")


# speedup vs baseline: 1.8874x; 1.8874x over previous
import functools

import jax
import jax.numpy as jnp
from jax import lax
from jax.experimental import pallas as pl
from jax.experimental.pallas import tpu as pltpu
from jax.experimental.pallas import tpu_sc as plsc

B = 16384
NW = 32
ROWS_PER_W = B // NW
CHUNK = 128
DG = 64
DM = 256


def _sc_gather_body(uidx, sidx, eu_gmf, eu_mlp, ei_gmf, ei_mlp,
                    ug_out, um_out, ig_out, im_out,
                    idx_u, idx_s, r_ug, r_um, r_ig, r_im, sem):
    wid = lax.axis_index("s") * 2 + lax.axis_index("c")
    base = wid * ROWS_PER_W
    for k in range(ROWS_PER_W // CHUNK):
        off = base + k * CHUNK
        pltpu.sync_copy(uidx.at[pl.ds(off, CHUNK)], idx_u)
        pltpu.sync_copy(sidx.at[pl.ds(off, CHUNK)], idx_s)
        h1 = pltpu.async_copy(eu_gmf.at[idx_u], r_ug, sem)
        h2 = pltpu.async_copy(eu_mlp.at[idx_u], r_um, sem)
        h3 = pltpu.async_copy(ei_gmf.at[idx_s], r_ig, sem)
        h4 = pltpu.async_copy(ei_mlp.at[idx_s], r_im, sem)
        h1.wait()
        h2.wait()
        h3.wait()
        h4.wait()
        pltpu.sync_copy(r_ug, ug_out.at[pl.ds(off, CHUNK)])
        pltpu.sync_copy(r_um, um_out.at[pl.ds(off, CHUNK)])
        pltpu.sync_copy(r_ig, ig_out.at[pl.ds(off, CHUNK)])
        pltpu.sync_copy(r_im, im_out.at[pl.ds(off, CHUNK)])


_sc_gather = pl.kernel(
    _sc_gather_body,
    mesh=plsc.VectorSubcoreMesh(core_axis_name="c", subcore_axis_name="s"),
    out_type=[
        jax.ShapeDtypeStruct((B, DG), jnp.float32),
        jax.ShapeDtypeStruct((B, DM), jnp.float32),
        jax.ShapeDtypeStruct((B, DG), jnp.float32),
        jax.ShapeDtypeStruct((B, DM), jnp.float32),
    ],
    scratch_types=[
        pltpu.VMEM((CHUNK,), jnp.int32),
        pltpu.VMEM((CHUNK,), jnp.int32),
        pltpu.VMEM((CHUNK, DG), jnp.float32),
        pltpu.VMEM((CHUNK, DM), jnp.float32),
        pltpu.VMEM((CHUNK, DG), jnp.float32),
        pltpu.VMEM((CHUNK, DM), jnp.float32),
        pltpu.SemaphoreType.DMA,
    ],
    compiler_params=pltpu.CompilerParams(use_tc_tiling_on_sc=False),
)


BBLK = 2048


def _tc_body(ug, um, ig, im, w1a, w1b, b1, w2, b2, w3, b3, wpa, wpb, bp, out):
    f32 = jnp.float32
    h = jnp.dot(um[...], w1a[...], preferred_element_type=f32)
    h += jnp.dot(im[...], w1b[...], preferred_element_type=f32)
    h = jnp.maximum(h + b1[...], 0.0)
    h = jnp.maximum(jnp.dot(h, w2[...], preferred_element_type=f32) + b2[...], 0.0)
    h = jnp.maximum(jnp.dot(h, w3[...], preferred_element_type=f32) + b3[...], 0.0)
    gmf = ug[...] * ig[...]
    pred = (jnp.sum(gmf * wpa[...], axis=-1, keepdims=True)
            + jnp.sum(h * wpb[...], axis=-1, keepdims=True)
            + bp[0, 0])
    out[...] = pred


def _tc_call(ug, um, ig, im, w1a, w1b, b1, w2, b2, w3, b3, wpa, wpb, bp):
    nblk = B // BBLK
    row = lambda i: (i, 0)
    rep = lambda i: (0, 0)
    return pl.pallas_call(
        _tc_body,
        grid=(nblk,),
        in_specs=[
            pl.BlockSpec((BBLK, DG), row),
            pl.BlockSpec((BBLK, DM), row),
            pl.BlockSpec((BBLK, DG), row),
            pl.BlockSpec((BBLK, DM), row),
            pl.BlockSpec((DM, DM), rep),
            pl.BlockSpec((DM, DM), rep),
            pl.BlockSpec((1, DM), rep),
            pl.BlockSpec((DM, 128), rep),
            pl.BlockSpec((1, 128), rep),
            pl.BlockSpec((128, DG), rep),
            pl.BlockSpec((1, DG), rep),
            pl.BlockSpec((1, DG), rep),
            pl.BlockSpec((1, DG), rep),
            pl.BlockSpec((1, 1), rep),
        ],
        out_specs=pl.BlockSpec((BBLK, 1), row),
        out_shape=jax.ShapeDtypeStruct((B, 1), jnp.float32),
        compiler_params=pltpu.CompilerParams(
            dimension_semantics=("parallel",)),
    )(ug, um, ig, im, w1a, w1b, b1, w2, b2, w3, b3, wpa, wpb, bp)


def kernel(userIdx, servIdx, eu_gmf, eu_mlp, ei_gmf, ei_mlp,
           W1, b1, W2, b2, W3, b3, Wp, bp):
    uidx = userIdx.astype(jnp.int32)
    sidx = servIdx.astype(jnp.int32)
    ug, um, ig, im = _sc_gather(uidx, sidx, eu_gmf, eu_mlp, ei_gmf, ei_mlp)
    w1a, w1b = W1[:DM], W1[DM:]
    wpa = Wp[:DG, 0].reshape(1, DG)
    wpb = Wp[DG:, 0].reshape(1, DG)
    out = _tc_call(ug, um, ig, im, w1a, w1b, b1.reshape(1, DM), W2,
                   b2.reshape(1, 128), W3, b3.reshape(1, DG),
                   wpa, wpb, bp.reshape(1, 1))
    return out.reshape(-1)
